# baseline (speedup 1.0000x reference)
import functools

import jax
import jax.numpy as jnp
from jax import lax
from jax.experimental import pallas as pl
from jax.experimental.pallas import tpu as pltpu
from jax.experimental.pallas import tpu_sc as plsc

N_TOK = 8192
K_CODES = 8192
D = 256
COMMIT = 0.25

T_BLK = 2048
K_BLK = 512
N_T = N_TOK // T_BLK
N_K = K_CODES // K_BLK

SC_NC = 2
SC_NS = 16
SC_NW = SC_NC * SC_NS
B_PER_W = N_TOK // SC_NW
G_CHUNK = 128

E_BLK = 1024
N_E = N_TOK // E_BLK


def _rna_bf16(v):
    u = lax.bitcast_convert_type(v, jnp.uint32)
    bump = jnp.where((u & jnp.uint32(0xFFFF)) >= jnp.uint32(0x8000),
                     jnp.uint32(0x10000), jnp.uint32(0))
    return lax.bitcast_convert_type((u & jnp.uint32(0xFFFF0000)) + bump,
                                    jnp.float32)


def _argmin_body(x_ref, e_ref, idx_ref, run_min, run_idx):
    k = pl.program_id(1)
    x = x_ref[...]
    e = e_ref[...]
    x2 = jnp.sum(x * x, axis=1, keepdims=True)
    e2 = jnp.sum(e * e, axis=1, keepdims=True)
    lhs = (2.0 * x).astype(jnp.bfloat16)
    rhs = _rna_bf16(e).astype(jnp.bfloat16)
    m = lax.dot_general(lhs, rhs, (((1,), (1,)), ((), ())),
                        preferred_element_type=jnp.float32)
    d = (x2 + e2.reshape(1, K_BLK)) - m
    tile_min = jnp.min(d, axis=1, keepdims=True)
    iota = lax.broadcasted_iota(jnp.int32, d.shape, 1)
    tile_arg = jnp.min(jnp.where(d == tile_min, iota, jnp.int32(2**30)),
                       axis=1, keepdims=True)

    @pl.when(k == 0)
    def _():
        run_min[...] = tile_min
        run_idx[...] = tile_arg

    @pl.when(k > 0)
    def _():
        better = tile_min < run_min[...]
        run_idx[...] = jnp.where(better, tile_arg + k * K_BLK, run_idx[...])
        run_min[...] = jnp.where(better, tile_min, run_min[...])

    @pl.when(k == N_K - 1)
    def _():
        idx_ref[...] = run_idx[...]


def _encode(flat, emb):
    return pl.pallas_call(
        _argmin_body,
        grid=(N_T, N_K),
        in_specs=[
            pl.BlockSpec((T_BLK, D), lambda t, k: (t, 0)),
            pl.BlockSpec((K_BLK, D), lambda t, k: (k, 0)),
        ],
        out_specs=pl.BlockSpec((T_BLK, 1), lambda t, k: (t, 0)),
        out_shape=jax.ShapeDtypeStruct((N_TOK, 1), jnp.int32),
        scratch_shapes=[
            pltpu.VMEM((T_BLK, 1), jnp.float32),
            pltpu.VMEM((T_BLK, 1), jnp.int32),
        ],
    )(flat, emb)


N_CHUNK = B_PER_W // G_CHUNK
ZERO_SL = K_CODES // SC_NS


def _sc_body(emb_hbm, idx_hbm, quant_hbm, pcounts_hbm,
             idx_v, rows_v, zeros_v, ones_v, counts_sh, sem):
    core = lax.axis_index("c")
    sid = lax.axis_index("s")
    wid = sid * SC_NC + core
    base = wid * B_PER_W
    pltpu.sync_copy(idx_hbm.at[wid], idx_v)

    def _zero(i, c):
        zeros_v[pl.ds(i * 16, 16)] = jnp.zeros((16,), jnp.int32)
        return c
    lax.fori_loop(0, ZERO_SL // 16, _zero, 0)

    def _one(i, c):
        ones_v[pl.ds(i * 16, 16)] = jnp.ones((16,), jnp.int32)
        return c
    lax.fori_loop(0, G_CHUNK // 16, _one, 0)

    pltpu.sync_copy(zeros_v, counts_sh.at[pl.ds(sid * ZERO_SL, ZERO_SL)])
    plsc.subcore_barrier()

    for c in range(N_CHUNK):
        pltpu.sync_copy(ones_v, counts_sh.at[idx_v.at[c]], add=True)
        pltpu.async_copy(emb_hbm.at[idx_v.at[c]], rows_v, sem).wait()
        pltpu.sync_copy(rows_v, quant_hbm.at[pl.ds(base + c * G_CHUNK, G_CHUNK)])

    plsc.subcore_barrier()

    @pl.when(sid == 0)
    def _():
        pltpu.sync_copy(counts_sh, pcounts_hbm.at[core])


def _sc_gather_hist(emb, idx):
    mesh = plsc.VectorSubcoreMesh(core_axis_name="c", subcore_axis_name="s")
    fn = pl.kernel(
        _sc_body,
        mesh=mesh,
        out_type=[
            jax.ShapeDtypeStruct((N_TOK, D), jnp.float32),
            jax.ShapeDtypeStruct((SC_NC, K_CODES), jnp.int32),
        ],
        scratch_types=[
            pltpu.VMEM((N_CHUNK, G_CHUNK), jnp.int32),
            pltpu.VMEM((G_CHUNK, D), jnp.float32),
            pltpu.VMEM((ZERO_SL,), jnp.int32),
            pltpu.VMEM((G_CHUNK,), jnp.int32),
            pltpu.VMEM_SHARED((K_CODES,), jnp.int32),
            pltpu.SemaphoreType.DMA,
        ],
    )
    return fn(emb, idx.reshape(SC_NW, N_CHUNK, G_CHUNK))


def _epi_body(x_ref, q_ref, pc_ref, st_ref, loss_ref, perp_ref, acc):
    i = pl.program_id(0)
    x = x_ref[...]
    q = q_ref[...]
    delta = q - x
    st_ref[...] = x + delta
    s = jnp.sum(delta * delta)

    @pl.when(i == 0)
    def _():
        acc[0, 0] = s

    @pl.when(i > 0)
    def _():
        acc[0, 0] = acc[0, 0] + s

    @pl.when(i == N_E - 1)
    def _():
        mean_sq = acc[0, 0] / jnp.float32(N_TOK * D)
        loss_ref[0, 0] = (1.0 + COMMIT) * mean_sq
        counts = jnp.sum(pc_ref[...], axis=0)
        avg = counts.astype(jnp.float32) * jnp.float32(1.0 / N_TOK)
        ent = jnp.sum(avg * jnp.log(avg + 1e-10))
        perp_ref[0, 0] = jnp.exp(-ent)


def _epilogue(flat, quant, pcounts):
    return pl.pallas_call(
        _epi_body,
        grid=(N_E,),
        in_specs=[
            pl.BlockSpec((E_BLK, D), lambda i: (i, 0)),
            pl.BlockSpec((E_BLK, D), lambda i: (i, 0)),
            pl.BlockSpec((SC_NC, K_CODES), lambda i: (0, 0)),
        ],
        out_specs=[
            pl.BlockSpec((E_BLK, D), lambda i: (i, 0)),
            pl.BlockSpec(memory_space=pltpu.SMEM),
            pl.BlockSpec(memory_space=pltpu.SMEM),
        ],
        out_shape=[
            jax.ShapeDtypeStruct((N_TOK, D), jnp.float32),
            jax.ShapeDtypeStruct((1, 1), jnp.float32),
            jax.ShapeDtypeStruct((1, 1), jnp.float32),
        ],
        scratch_shapes=[pltpu.SMEM((1, 1), jnp.float32)],
    )(flat, quant, pcounts)


def kernel(inputs, embedding_weight):
    flat = inputs.reshape(-1, D)
    idx = _encode(flat, embedding_weight).reshape(-1)
    quant, pcounts = _sc_gather_hist(embedding_weight, idx)
    st, loss, perp = _epilogue(flat, quant, pcounts)
    return st.reshape(inputs.shape), loss[0, 0], perp[0, 0]
